# baseline (speedup 1.0000x reference)
import functools

import jax
import jax.numpy as jnp
from jax import lax
from jax.experimental import pallas as pl
from jax.experimental.pallas import tpu as pltpu
from jax.experimental.pallas import tpu_sc as plsc

V = 100000
D = 64
NC = 2
NS = 16
NW = NC * NS

BLK = 80
SLOTS = 8
LOOKAHEAD = 4

SENT_ROWS = 51200
CHAR_ROWS = 1024000
NB_SENT = SENT_ROWS // (NW * BLK)
NB_CHAR = CHAR_ROWS // (NW * BLK)

ROWS_PER_W = V // NW
MCHUNK = 625 * D

_mesh = plsc.VectorSubcoreMesh(
    core_axis_name="c", subcore_axis_name="s", num_cores=NC, num_subcores=NS
)


@functools.partial(
    pl.kernel,
    out_type=jax.ShapeDtypeStruct((V * D,), jnp.float32),
    mesh=_mesh,
    scratch_types=[
        pltpu.VMEM((MCHUNK,), jnp.float32),
        pltpu.VMEM((MCHUNK,), jnp.float32),
        pltpu.SemaphoreType.DMA((2,)),
        pltpu.SemaphoreType.DMA((2,)),
    ],
)
def _mask_table(emb, out, buf0, buf1, isem, osem):
    wid = lax.axis_index("s") * NC + lax.axis_index("c")
    base = pl.multiple_of(wid * (ROWS_PER_W * D), 8)
    bufs = (buf0, buf1)
    nch = (ROWS_PER_W * D) // MCHUNK

    def in_cp(c):
        return pltpu.make_async_copy(
            emb.at[pl.ds(base + c * MCHUNK, MCHUNK)], bufs[c % 2], isem.at[c % 2]
        )

    def out_cp(c):
        return pltpu.make_async_copy(
            bufs[c % 2], out.at[pl.ds(base + c * MCHUNK, MCHUNK)], osem.at[c % 2]
        )

    in_cp(0).start()
    for c in range(nch):
        in_cp(c).wait()
        if c == 0:
            @pl.when(wid == 0)
            def _():
                zf = jnp.zeros((16,), jnp.float32)
                for cc in range(D // 16):
                    buf0[pl.ds(cc * 16, 16)] = zf
        out_cp(c).start()
        if c + 1 < nch:
            if c >= 1:
                out_cp(c - 1).wait()
            in_cp(c + 1).start()
    out_cp(nch - 2).wait()
    out_cp(nch - 1).wait()


def _stream(table, idx_hbm, out_hbm, idx_v, rows, gsem, wsem, wid, nblk, sent3d):
    base = wid * nblk
    ibase = pl.multiple_of(base * BLK, 8)
    pltpu.sync_copy(idx_hbm.at[pl.ds(ibase, nblk * BLK)], idx_v.at[pl.ds(0, nblk * BLK)])

    def gather_cp(i, slot):
        return pltpu.make_async_copy(
            table.at[idx_v.at[pl.ds(i * BLK, BLK)]], rows.at[slot], gsem.at[slot]
        )

    def wb_cps(i, slot):
        if sent3d:
            w0 = (base + i) * (BLK // 20)
            return [
                pltpu.make_async_copy(
                    rows.at[slot, pl.ds(20 * k, 20)], out_hbm.at[w0 + k], wsem.at[slot]
                )
                for k in range(BLK // 20)
            ]
        return [
            pltpu.make_async_copy(
                rows.at[slot],
                out_hbm.at[pl.ds(pl.multiple_of((base + i) * BLK, 8), BLK)],
                wsem.at[slot],
            )
        ]

    def prologue(b, carry):
        gather_cp(b, b % SLOTS).start()
        return carry

    lax.fori_loop(0, LOOKAHEAD, prologue, 0)

    def body(i, carry):
        slot = i % SLOTS
        gather_cp(i, slot).wait()
        for cp in wb_cps(i, slot):
            cp.start()
        j = i + LOOKAHEAD

        @pl.when(j < nblk)
        def _():
            @pl.when(j >= SLOTS)
            def _():
                for cp in wb_cps(j - SLOTS, j % SLOTS):
                    cp.wait()

            gather_cp(j, j % SLOTS).start()

        return carry

    lax.fori_loop(0, nblk, body, 0)

    def drain(i, carry):
        for cp in wb_cps(i, i % SLOTS):
            cp.wait()
        return carry

    lax.fori_loop(max(nblk - SLOTS, 0), nblk, drain, 0)


@functools.partial(
    pl.kernel,
    out_type=(
        jax.ShapeDtypeStruct((SENT_ROWS, D), jnp.float32),
        jax.ShapeDtypeStruct((SENT_ROWS, 20, D), jnp.float32),
        jax.ShapeDtypeStruct((SENT_ROWS, 20, D), jnp.float32),
    ),
    mesh=_mesh,
    scratch_types=[
        pltpu.VMEM((NB_CHAR * BLK,), jnp.int32),
        pltpu.VMEM((SLOTS, BLK, D), jnp.float32),
        pltpu.SemaphoreType.DMA((SLOTS,)),
        pltpu.SemaphoreType.DMA((SLOTS,)),
    ],
    compiler_params=pltpu.CompilerParams(use_tc_tiling_on_sc=False),
)
def _gather_all(table, sidx, cidx, nidx, sout, cout, nout, idx_v, rows, gsem, wsem):
    wid = lax.axis_index("s") * NC + lax.axis_index("c")
    _stream(table, sidx, sout, idx_v, rows, gsem, wsem, wid, NB_SENT, False)
    _stream(table, cidx, cout, idx_v, rows, gsem, wsem, wid, NB_CHAR, True)
    _stream(table, nidx, nout, idx_v, rows, gsem, wsem, wid, NB_CHAR, True)


def kernel(sent_input, char_input, n_gram_input, emb):
    si = sent_input.astype(jnp.int32).reshape(SENT_ROWS)
    ci = char_input.astype(jnp.int32).reshape(CHAR_ROWS)
    ni = n_gram_input.astype(jnp.int32).reshape(CHAR_ROWS)
    emb_flat = emb.astype(jnp.float32).reshape(V * D)
    masked = _mask_table(emb_flat).reshape(V, D)
    s, c, n = _gather_all(masked, si, ci, ni)
    return (s.reshape(1024, 50, D), c, n)


# speedup vs baseline: 1.3403x; 1.3403x over previous
import functools

import jax
import jax.numpy as jnp
from jax import lax
from jax.experimental import pallas as pl
from jax.experimental.pallas import tpu as pltpu
from jax.experimental.pallas import tpu_sc as plsc

V = 100000
D = 64
NC = 2
NS = 16
NW = NC * NS

BLK = 80
SLOTS = 8
LOOKAHEAD = 4

SENT_ROWS = 51200
CHAR_ROWS = 1024000
NB_SENT = SENT_ROWS // (NW * BLK)
NB_CHAR = CHAR_ROWS // (NW * BLK)

ROWS_PER_W = V // NW
MCHUNK = 625 * D

_mesh = plsc.VectorSubcoreMesh(
    core_axis_name="c", subcore_axis_name="s", num_cores=NC, num_subcores=NS
)


@functools.partial(
    pl.kernel,
    out_type=jax.ShapeDtypeStruct((V * D,), jnp.float32),
    mesh=_mesh,
    scratch_types=[
        pltpu.VMEM((MCHUNK,), jnp.float32),
        pltpu.VMEM((MCHUNK,), jnp.float32),
        pltpu.SemaphoreType.DMA((2,)),
        pltpu.SemaphoreType.DMA((2,)),
    ],
)
def _mask_table(emb, out, buf0, buf1, isem, osem):
    wid = lax.axis_index("s") * NC + lax.axis_index("c")
    base = pl.multiple_of(wid * (ROWS_PER_W * D), 8)
    bufs = (buf0, buf1)
    nch = (ROWS_PER_W * D) // MCHUNK

    def in_cp(c):
        return pltpu.make_async_copy(
            emb.at[pl.ds(base + c * MCHUNK, MCHUNK)], bufs[c % 2], isem.at[c % 2]
        )

    def out_cp(c):
        return pltpu.make_async_copy(
            bufs[c % 2], out.at[pl.ds(base + c * MCHUNK, MCHUNK)], osem.at[c % 2]
        )

    in_cp(0).start()
    for c in range(nch):
        in_cp(c).wait()
        if c == 0:
            @pl.when(wid == 0)
            def _():
                zf = jnp.zeros((16,), jnp.float32)
                for cc in range(D // 16):
                    buf0[pl.ds(cc * 16, 16)] = zf
        out_cp(c).start()
        if c + 1 < nch:
            if c >= 1:
                out_cp(c - 1).wait()
            in_cp(c + 1).start()
    out_cp(nch - 2).wait()
    out_cp(nch - 1).wait()


def _stream(table, idx_hbm, out_hbm, idx_v, rows, gsem, wsem, wid, nblk):
    base = wid * nblk
    ibase = pl.multiple_of(base * BLK, 8)
    pltpu.sync_copy(idx_hbm.at[pl.ds(ibase, nblk * BLK)], idx_v.at[pl.ds(0, nblk * BLK)])

    def gather_cp(i, slot):
        return pltpu.make_async_copy(
            table.at[idx_v.at[pl.ds(i * BLK, BLK)]], rows.at[slot], gsem.at[slot]
        )

    def wb_cp(i, slot):
        return pltpu.make_async_copy(
            rows.at[slot],
            out_hbm.at[pl.ds(pl.multiple_of((base + i) * BLK, 8), BLK)],
            wsem.at[slot],
        )

    def prologue(b, carry):
        gather_cp(b, b % SLOTS).start()
        return carry

    lax.fori_loop(0, LOOKAHEAD, prologue, 0)

    def body(i, carry):
        slot = i % SLOTS
        gather_cp(i, slot).wait()
        wb_cp(i, slot).start()
        j = i + LOOKAHEAD

        @pl.when(j < nblk)
        def _():
            @pl.when(j >= SLOTS)
            def _():
                wb_cp(j - SLOTS, j % SLOTS).wait()

            gather_cp(j, j % SLOTS).start()

        return carry

    lax.fori_loop(0, nblk, body, 0)

    def drain(i, carry):
        wb_cp(i, i % SLOTS).wait()
        return carry

    lax.fori_loop(max(nblk - SLOTS, 0), nblk, drain, 0)


@functools.partial(
    pl.kernel,
    out_type=(
        jax.ShapeDtypeStruct((SENT_ROWS, D), jnp.float32),
        jax.ShapeDtypeStruct((CHAR_ROWS, D), jnp.float32),
        jax.ShapeDtypeStruct((CHAR_ROWS, D), jnp.float32),
    ),
    mesh=_mesh,
    scratch_types=[
        pltpu.VMEM((NB_CHAR * BLK,), jnp.int32),
        pltpu.VMEM((SLOTS, BLK, D), jnp.float32),
        pltpu.SemaphoreType.DMA((SLOTS,)),
        pltpu.SemaphoreType.DMA((SLOTS,)),
    ],
    compiler_params=pltpu.CompilerParams(use_tc_tiling_on_sc=False),
)
def _gather_all(table, sidx, cidx, nidx, sout, cout, nout, idx_v, rows, gsem, wsem):
    wid = lax.axis_index("s") * NC + lax.axis_index("c")
    _stream(table, sidx, sout, idx_v, rows, gsem, wsem, wid, NB_SENT)
    _stream(table, cidx, cout, idx_v, rows, gsem, wsem, wid, NB_CHAR)
    _stream(table, nidx, nout, idx_v, rows, gsem, wsem, wid, NB_CHAR)


def _make_transpose(n_items, rpi, wblk=128):
    P = (rpi * D) // 128

    def body(in_ref, out_ref):
        x = in_ref[...].reshape(wblk, P, 128)
        for p in range(P):
            out_ref[p] = x[:, p, :].T

    return pl.pallas_call(
        body,
        grid=(n_items // wblk,),
        in_specs=[pl.BlockSpec((wblk * P, 128), lambda i: (i, 0))],
        out_specs=pl.BlockSpec((P, 128, wblk), lambda i: (0, 0, i)),
        out_shape=jax.ShapeDtypeStruct((P, 128, n_items), jnp.float32),
    )


_tr_sent = _make_transpose(1024, 50)
_tr_char = _make_transpose(SENT_ROWS, 20)


def kernel(sent_input, char_input, n_gram_input, emb):
    si = sent_input.astype(jnp.int32).reshape(SENT_ROWS)
    ci = char_input.astype(jnp.int32).reshape(CHAR_ROWS)
    ni = n_gram_input.astype(jnp.int32).reshape(CHAR_ROWS)
    emb_flat = emb.astype(jnp.float32).reshape(V * D)
    masked = _mask_table(emb_flat).reshape(V, D)
    s, c, n = _gather_all(masked, si, ci, ni)
    st = _tr_sent(s.reshape(SENT_ROWS * D // 128, 128))
    ct = _tr_char(c.reshape(CHAR_ROWS * D // 128, 128))
    nt = _tr_char(n.reshape(CHAR_ROWS * D // 128, 128))
    return (
        jnp.transpose(st.reshape(50, D, 1024), (2, 0, 1)),
        jnp.transpose(ct.reshape(20, D, SENT_ROWS), (2, 0, 1)),
        jnp.transpose(nt.reshape(20, D, SENT_ROWS), (2, 0, 1)),
    )
